# baseline (speedup 1.0000x reference)
import functools

import jax
import jax.numpy as jnp
from jax import lax
from jax.experimental import pallas as pl
from jax.experimental.pallas import tpu as pltpu
from jax.experimental.pallas import tpu_sc as plsc

_V = 1000000
_D = 32
_B = 16384 * 50

_NC = 2
_NS = 16
_NW = _NC * _NS
_BPW = _B // _NW
_CHUNK = 800
_NCH = _BPW // _CHUNK
_NBUF = 4
_LOOK = _NBUF - 1


_mesh = plsc.VectorSubcoreMesh(core_axis_name="c", subcore_axis_name="s")


@functools.partial(
    pl.kernel,
    mesh=_mesh,
    out_type=jax.ShapeDtypeStruct((_B, _D), jnp.float32),
    scratch_types=[
        pltpu.VMEM((_BPW,), jnp.int32),
        pltpu.VMEM((_NBUF, _CHUNK, _D), jnp.float32),
        pltpu.SemaphoreType.DMA((_NBUF,)),
        pltpu.SemaphoreType.DMA((_NBUF,)),
    ],
    compiler_params=pltpu.CompilerParams(use_tc_tiling_on_sc=False),
)
def _emb_lookup(x_hbm, table_hbm, out_hbm, idx_v, rows_v, gsem, ssem):
    wid = lax.axis_index("s") * _NC + lax.axis_index("c")
    base = wid * _BPW

    pltpu.sync_copy(x_hbm.at[pl.ds(base, _BPW)], idx_v)

    def gather(i, b):
        return pltpu.make_async_copy(
            table_hbm.at[idx_v.at[pl.ds(i * _CHUNK, _CHUNK)]],
            rows_v.at[b],
            gsem.at[b],
        )

    def store(i, b):
        return pltpu.make_async_copy(
            rows_v.at[b],
            out_hbm.at[pl.ds(base + i * _CHUNK, _CHUNK)],
            ssem.at[b],
        )

    for b in range(_LOOK):
        gather(b, b).start()

    def outer(g, carry):
        for b in range(_NBUF):
            i = g * _NBUF + b
            gather(i, b).wait()

            bp = (b - 1) % _NBUF

            @pl.when(i >= 1)
            def _():
                store(i - 1, bp).wait()

            bn = (b + _LOOK) % _NBUF

            @pl.when(i + _LOOK < _NCH)
            def _():
                gather(i + _LOOK, bn).start()

            store(i, b).start()
        return carry

    lax.fori_loop(0, _NCH // _NBUF, outer, 0)
    store(_NCH - 1, (_NCH - 1) % _NBUF).wait()


_S = 50
_NB = 16384


_TCHUNK = 8192


_TRG = (_V + _TCHUNK - 1) // _TCHUNK
_TREM = _V - (_TRG - 1) * _TCHUNK


def _tr_body(t_ref, o_hbm, buf, sem):
    j = pl.program_id(0)
    buf[...] = t_ref[...].T

    @pl.when(j < _TRG - 1)
    def _():
        cp = pltpu.make_async_copy(
            buf, o_hbm.at[pl.ds(j * _TCHUNK, _TCHUNK), :], sem)
        cp.start()
        cp.wait()

    @pl.when(j == _TRG - 1)
    def _():
        cp = pltpu.make_async_copy(
            buf.at[pl.ds(0, _TREM), :],
            o_hbm.at[pl.ds(j * _TCHUNK, _TREM), :], sem)
        cp.start()
        cp.wait()


def _detile_table(tT):
    return pl.pallas_call(
        _tr_body,
        grid=(_TRG,),
        in_specs=[pl.BlockSpec((_D, _TCHUNK), lambda j: (0, j))],
        out_specs=pl.BlockSpec(memory_space=pl.ANY),
        out_shape=jax.ShapeDtypeStruct((_V, _D), jnp.float32),
        scratch_shapes=[
            pltpu.VMEM((_TCHUNK, _D), jnp.float32),
            pltpu.SemaphoreType.DMA,
        ],
    )(tT)


def kernel(x, table):
    flat_idx = jnp.transpose(x).reshape(-1).astype(jnp.int32)
    tbl = _detile_table(jnp.transpose(table))
    out = _emb_lookup(flat_idx, tbl)
    return out.reshape(_S, _NB, _D).transpose(1, 0, 2)


# speedup vs baseline: 1.7856x; 1.7856x over previous
import functools

import jax
import jax.numpy as jnp
from jax import lax
from jax.experimental import pallas as pl
from jax.experimental.pallas import tpu as pltpu
from jax.experimental.pallas import tpu_sc as plsc

_V = 1000000
_D = 32
_DP = 128
_B = 16384 * 50

_NC = 2
_NS = 16
_NW = _NC * _NS
_BPW = _B // _NW
_CHUNK = 200
_NCH = _BPW // _CHUNK
_NBUF = 4
_LOOK = _NBUF - 1


_mesh = plsc.VectorSubcoreMesh(core_axis_name="c", subcore_axis_name="s")


@functools.partial(
    pl.kernel,
    mesh=_mesh,
    out_type=jax.ShapeDtypeStruct((_B, _DP), jnp.float32),
    scratch_types=[
        pltpu.VMEM((_BPW,), jnp.int32),
        pltpu.VMEM((_NBUF, _CHUNK, _DP), jnp.float32),
        pltpu.SemaphoreType.DMA((_NBUF,)),
        pltpu.SemaphoreType.DMA((_NBUF,)),
    ],
    compiler_params=pltpu.CompilerParams(use_tc_tiling_on_sc=False),
)
def _emb_lookup(x_hbm, table_hbm, out_hbm, idx_v, rows_v, gsem, ssem):
    wid = lax.axis_index("s") * _NC + lax.axis_index("c")
    base = wid * _BPW

    pltpu.sync_copy(x_hbm.at[pl.ds(base, _BPW)], idx_v)

    def gather(i, b):
        return pltpu.make_async_copy(
            table_hbm.at[idx_v.at[pl.ds(i * _CHUNK, _CHUNK)]],
            rows_v.at[b],
            gsem.at[b],
        )

    def store(i, b):
        return pltpu.make_async_copy(
            rows_v.at[b],
            out_hbm.at[pl.ds(base + i * _CHUNK, _CHUNK)],
            ssem.at[b],
        )

    for b in range(_LOOK):
        gather(b, b).start()

    def outer(g, carry):
        for b in range(_NBUF):
            i = g * _NBUF + b
            gather(i, b).wait()

            bp = (b - 1) % _NBUF

            @pl.when(i >= 1)
            def _():
                store(i - 1, bp).wait()

            bn = (b + _LOOK) % _NBUF

            @pl.when(i + _LOOK < _NCH)
            def _():
                gather(i + _LOOK, bn).start()

            store(i, b).start()
        return carry

    lax.fori_loop(0, _NCH // _NBUF, outer, 0)
    store(_NCH - 1, (_NCH - 1) % _NBUF).wait()


_S = 50
_NB = 16384

_TCHUNK = 8192
_TRG = (_V + _TCHUNK - 1) // _TCHUNK


def _tr_body(t_ref, o_ref):
    o_ref[:, : _D] = t_ref[...].T


def _detile_table(tT):
    return pl.pallas_call(
        _tr_body,
        grid=(_TRG,),
        in_specs=[pl.BlockSpec((_D, _TCHUNK), lambda j: (0, j))],
        out_specs=pl.BlockSpec((_TCHUNK, _DP), lambda j: (j, 0)),
        out_shape=jax.ShapeDtypeStruct((_V, _DP), jnp.float32),
    )(tT)


def kernel(x, table):
    flat_idx = jnp.transpose(x).reshape(-1).astype(jnp.int32)
    tbl = _detile_table(jnp.transpose(table))
    out = _emb_lookup(flat_idx, tbl)
    return out.reshape(_S, _NB, _DP)[:, :, :_D].transpose(1, 0, 2)


# speedup vs baseline: 1.8766x; 1.0509x over previous
import functools

import jax
import jax.numpy as jnp
from jax import lax
from jax.experimental import pallas as pl
from jax.experimental.pallas import tpu as pltpu
from jax.experimental.pallas import tpu_sc as plsc

_V = 1000000
_D = 32
_DP = 128
_B = 16384 * 50

_NC = 2
_NS = 16
_NW = _NC * _NS
_BPW = _B // _NW
_CHUNK = 200
_NCH = _BPW // _CHUNK
_NBUF = 4
_LOOK = _NBUF - 1


_mesh = plsc.VectorSubcoreMesh(core_axis_name="c", subcore_axis_name="s")


@functools.partial(
    pl.kernel,
    mesh=_mesh,
    out_type=jax.ShapeDtypeStruct((_B, _DP), jnp.float32),
    scratch_types=[
        pltpu.VMEM((_BPW,), jnp.int32),
        pltpu.VMEM((_NBUF, _CHUNK, _DP), jnp.float32),
        pltpu.SemaphoreType.DMA((_NBUF,)),
        pltpu.SemaphoreType.DMA((_NBUF,)),
    ],
    compiler_params=pltpu.CompilerParams(use_tc_tiling_on_sc=False),
)
def _emb_lookup(x_hbm, table_hbm, out_hbm, idx_v, rows_v, gsem, ssem):
    wid = lax.axis_index("s") * _NC + lax.axis_index("c")
    base = wid * _BPW

    pltpu.sync_copy(x_hbm.at[pl.ds(base, _BPW)], idx_v)

    def gather(i, b):
        return pltpu.make_async_copy(
            table_hbm.at[idx_v.at[pl.ds(i * _CHUNK, _CHUNK)]],
            rows_v.at[b],
            gsem.at[b],
        )

    def store(i, b):
        return pltpu.make_async_copy(
            rows_v.at[b],
            out_hbm.at[pl.ds(base + i * _CHUNK, _CHUNK)],
            ssem.at[b],
        )

    for b in range(_LOOK):
        gather(b, b).start()

    def outer(g, carry):
        for b in range(_NBUF):
            i = g * _NBUF + b
            gather(i, b).wait()

            bp = (b - 1) % _NBUF

            @pl.when(i >= 1)
            def _():
                store(i - 1, bp).wait()

            bn = (b + _LOOK) % _NBUF

            @pl.when(i + _LOOK < _NCH)
            def _():
                gather(i + _LOOK, bn).start()

            store(i, b).start()
        return carry

    lax.fori_loop(0, _NCH // _NBUF, outer, 0)
    store(_NCH - 1, (_NCH - 1) % _NBUF).wait()


_S = 50
_NB = 16384

_TCHUNK = 16384
_TRG = (_V + _TCHUNK - 1) // _TCHUNK


def _tr_body(t_ref, o_ref):
    o_ref[:, : _D] = t_ref[...].T


def _detile_table(tT):
    return pl.pallas_call(
        _tr_body,
        grid=(_TRG,),
        in_specs=[pl.BlockSpec((_D, _TCHUNK), lambda j: (0, j))],
        out_specs=pl.BlockSpec((_TCHUNK, _DP), lambda j: (j, 0)),
        out_shape=jax.ShapeDtypeStruct((_V, _DP), jnp.float32),
    )(tT)


def kernel(x, table):
    flat_idx = jnp.transpose(x).reshape(-1).astype(jnp.int32)
    tbl = _detile_table(jnp.transpose(table))
    out = _emb_lookup(flat_idx, tbl)
    return out.reshape(_S, _NB, _DP)[:, :, :_D].transpose(1, 0, 2)


# speedup vs baseline: 1.8871x; 1.0056x over previous
import functools

import jax
import jax.numpy as jnp
from jax import lax
from jax.experimental import pallas as pl
from jax.experimental.pallas import tpu as pltpu
from jax.experimental.pallas import tpu_sc as plsc

_V = 1000000
_D = 32
_DP = 128
_B = 16384 * 50

_NC = 2
_NS = 16
_NW = _NC * _NS
_BPW = _B // _NW
_CHUNK = 200
_NCH = _BPW // _CHUNK
_NBUF = 4
_LOOK = _NBUF - 1


_mesh = plsc.VectorSubcoreMesh(core_axis_name="c", subcore_axis_name="s")


@functools.partial(
    pl.kernel,
    mesh=_mesh,
    out_type=jax.ShapeDtypeStruct((_B, _DP), jnp.float32),
    scratch_types=[
        pltpu.VMEM((_BPW,), jnp.int32),
        pltpu.VMEM((_NBUF, _CHUNK, _DP), jnp.float32),
        pltpu.SemaphoreType.DMA((_NBUF,)),
        pltpu.SemaphoreType.DMA((_NBUF,)),
    ],
    compiler_params=pltpu.CompilerParams(use_tc_tiling_on_sc=False),
)
def _emb_lookup(x_hbm, table_hbm, out_hbm, idx_v, rows_v, gsem, ssem):
    wid = lax.axis_index("s") * _NC + lax.axis_index("c")
    base = wid * _BPW

    pltpu.sync_copy(x_hbm.at[pl.ds(base, _BPW)], idx_v)

    def gather(i, b):
        return pltpu.make_async_copy(
            table_hbm.at[idx_v.at[pl.ds(i * _CHUNK, _CHUNK)]],
            rows_v.at[b],
            gsem.at[b],
        )

    def store(i, b):
        return pltpu.make_async_copy(
            rows_v.at[b],
            out_hbm.at[pl.ds(base + i * _CHUNK, _CHUNK)],
            ssem.at[b],
        )

    for b in range(_LOOK):
        gather(b, b).start()

    def outer(g, carry):
        for b in range(_NBUF):
            i = g * _NBUF + b
            gather(i, b).wait()

            bp = (b - 1) % _NBUF

            @pl.when(i >= 1)
            def _():
                store(i - 1, bp).wait()

            bn = (b + _LOOK) % _NBUF

            @pl.when(i + _LOOK < _NCH)
            def _():
                gather(i + _LOOK, bn).start()

            store(i, b).start()
        return carry

    lax.fori_loop(0, _NCH // _NBUF, outer, 0)
    store(_NCH - 1, (_NCH - 1) % _NBUF).wait()


_S = 50
_NB = 16384

_TCHUNK = 32768
_TRG = (_V + _TCHUNK - 1) // _TCHUNK


def _tr_body(t_ref, o_ref):
    o_ref[:, : _D] = t_ref[...].T


def _detile_table(tT):
    return pl.pallas_call(
        _tr_body,
        grid=(_TRG,),
        in_specs=[pl.BlockSpec((_D, _TCHUNK), lambda j: (0, j))],
        out_specs=pl.BlockSpec((_TCHUNK, _DP), lambda j: (j, 0)),
        out_shape=jax.ShapeDtypeStruct((_V, _DP), jnp.float32),
    )(tT)


def kernel(x, table):
    flat_idx = jnp.transpose(x).reshape(-1).astype(jnp.int32)
    tbl = _detile_table(jnp.transpose(table))
    out = _emb_lookup(flat_idx, tbl)
    return out.reshape(_S, _NB, _DP)[:, :, :_D].transpose(1, 0, 2)
